# Initial kernel scaffold; baseline (speedup 1.0000x reference)
import jax
import jax.numpy as jnp
from jax.experimental import pallas as pl


def kernel(enc_h, y, embed_table):
    raise NotImplementedError("write your pallas kernel here")


# speedup vs baseline: 6.3462x; 6.3462x over previous
import functools

import jax
import jax.numpy as jnp
from jax import lax
from jax.experimental import pallas as pl
from jax.experimental.pallas import tpu as pltpu
from jax.experimental.pallas import tpu_sc as plsc

ALPHABET_SIZE = 100000
EMBED_DIM = 128
BATCH = 4096
SEQ = 200

N = BATCH * SEQ
NC = 2
NS = 16
NW = NC * NS
PER_W = N // NW
CHUNK = 128
NCHUNK = PER_W // CHUNK


def _sc_gather(y_r, table):
    mesh = plsc.VectorSubcoreMesh(core_axis_name="c", subcore_axis_name="s")

    @functools.partial(
        pl.kernel,
        mesh=mesh,
        out_type=jax.ShapeDtypeStruct((N, EMBED_DIM), jnp.float32),
        scratch_types=[
            pltpu.VMEM((NCHUNK, CHUNK), jnp.int32),
            pltpu.VMEM((CHUNK, EMBED_DIM), jnp.float32),
            pltpu.SemaphoreType.DMA,
        ],
    )
    def k(y_hbm, table_hbm, out_hbm, idx_v, rows_v, gsem):
        wid = lax.axis_index("s") * NC + lax.axis_index("c")
        base = wid * PER_W
        pltpu.sync_copy(y_hbm.at[wid], idx_v)

        def body(c, carry):
            pltpu.async_copy(table_hbm.at[idx_v.at[c]], rows_v, gsem).wait()
            pltpu.sync_copy(rows_v, out_hbm.at[pl.ds(base + c * CHUNK, CHUNK)])
            return carry

        lax.fori_loop(0, NCHUNK, body, 0)

    return k(y_r, table)


def kernel(enc_h, y, embed_table):
    del enc_h
    y_r = y.reshape(NW, NCHUNK, CHUNK)
    out = _sc_gather(y_r, embed_table)
    return out.reshape(BATCH, SEQ, EMBED_DIM)


# speedup vs baseline: 9.1031x; 1.4344x over previous
import functools

import jax
import jax.numpy as jnp
from jax import lax
from jax.experimental import pallas as pl
from jax.experimental.pallas import tpu as pltpu
from jax.experimental.pallas import tpu_sc as plsc

ALPHABET_SIZE = 100000
EMBED_DIM = 128
BATCH = 4096
SEQ = 200

N = BATCH * SEQ
NC = 2
NS = 16
NW = NC * NS
PER_W = N // NW
CHUNK = 128
NCHUNK = PER_W // CHUNK
NBUF = 4
NOUTER = NCHUNK // NBUF


def _sc_gather(y_r, table):
    mesh = plsc.VectorSubcoreMesh(core_axis_name="c", subcore_axis_name="s")

    @functools.partial(
        pl.kernel,
        mesh=mesh,
        out_type=jax.ShapeDtypeStruct((N, EMBED_DIM), jnp.float32),
        scratch_types=[
            pltpu.VMEM((NCHUNK, CHUNK), jnp.int32),
            pltpu.VMEM((NBUF, CHUNK, EMBED_DIM), jnp.float32),
        ] + [pltpu.SemaphoreType.DMA] * (2 * NBUF),
    )
    def k(y_hbm, table_hbm, out_hbm, idx_v, rows_v, *sems):
        gsem, osem = sems[:NBUF], sems[NBUF:]
        wid = lax.axis_index("s") * NC + lax.axis_index("c")
        base = wid * PER_W
        pltpu.sync_copy(y_hbm.at[wid], idx_v)

        def gather_wait(b):
            pltpu.make_async_copy(
                table_hbm.at[idx_v.at[0]], rows_v.at[b], gsem[b]).wait()

        def write_wait(b):
            pltpu.make_async_copy(
                rows_v.at[b], out_hbm.at[pl.ds(base, CHUNK)], osem[b]).wait()

        for b in range(NBUF):
            pltpu.async_copy(
                table_hbm.at[idx_v.at[b]], rows_v.at[b], gsem[b])

        def outer(g, carry):
            c0 = g * NBUF
            for b in range(NBUF):
                gather_wait(b)
                pltpu.async_copy(
                    rows_v.at[b],
                    out_hbm.at[pl.ds(base + (c0 + b) * CHUNK, CHUNK)],
                    osem[b])

            @pl.when(g < NOUTER - 1)
            def _():
                for b in range(NBUF):
                    write_wait(b)
                    pltpu.async_copy(
                        table_hbm.at[idx_v.at[c0 + NBUF + b]],
                        rows_v.at[b], gsem[b])

            return carry

        lax.fori_loop(0, NOUTER, outer, 0)
        for b in range(NBUF):
            write_wait(b)

    return k(y_r, table)


def kernel(enc_h, y, embed_table):
    del enc_h
    y_r = y.reshape(NW, NCHUNK, CHUNK)
    out = _sc_gather(y_r, embed_table)
    return out.reshape(BATCH, SEQ, EMBED_DIM)
